# baseline (speedup 1.0000x reference)
import functools

import jax
import jax.numpy as jnp
from jax import lax
from jax.experimental import pallas as pl
from jax.experimental.pallas import tpu as pltpu
from jax.experimental.pallas import tpu_sc as plsc

BATCH = 16384
EMBED_DIM = 64
AVG_RATING = 3.0

_NC = 2
_NS = 16
_NW = _NC * _NS
_BPW = BATCH // _NW
_CHUNK = 128
_NCHUNK = _BPW // _CHUNK
_GROUPS = _CHUNK // 16


def _body(user_hbm, item_hbm, ue2_hbm, ie2_hbm,
          user_bias_hbm, item_bias_hbm,
          out_hbm, ub_hbm, ib_hbm,
          idx_u, idx_i, idxh, u2, i2, ub_v, ib_v, out_v, sem, semb):
    wid = lax.axis_index("s") * _NC + lax.axis_index("c")
    base = wid * _BPW

    for k in range(_NCHUNK):
        pltpu.sync_copy(user_hbm.at[pl.ds(base + k * _CHUNK, _CHUNK)], idx_u.at[k])
        pltpu.sync_copy(item_hbm.at[pl.ds(base + k * _CHUNK, _CHUNK)], idx_i.at[k])

    bias_copies = []
    for k in range(_NCHUNK):
        sl = pl.ds(k * _CHUNK, _CHUNK)
        bias_copies.append(pltpu.async_copy(user_bias_hbm.at[idx_u.at[k]], ub_v.at[sl], semb))
        bias_copies.append(pltpu.async_copy(item_bias_hbm.at[idx_i.at[k]], ib_v.at[sl], semb))

    for k in range(_NCHUNK):
        for s in range(_CHUNK // 16):
            sl = pl.ds(s * 16, 16)
            idxh[2 * k, sl] = lax.shift_right_logical(idx_u[k, sl], 1)
            idxh[2 * k + 1, sl] = lax.shift_right_logical(idx_i[k, sl], 1)

    lane = lax.iota(jnp.int32, 16)

    def chunk_compute(k, uref, iref):
        def group(g, carry):
            gsl = pl.ds(g * 16, 16)
            pu_vec = (idx_u[k, gsl] & 1) * 64
            pi_vec = (idx_i[k, gsl] & 1) * 64
            vecs = []
            for j in range(16):
                r = g * 16 + j
                pu = pu_vec[j]
                pi = pi_vec[j]
                acc = None
                for t in range(EMBED_DIM // 16):
                    uv = uref[r, pl.ds(pu + t * 16, 16)]
                    iv = iref[r, pl.ds(pi + t * 16, 16)]
                    acc = uv * iv if acc is None else acc + uv * iv
                vecs.append(acc)
            sh = 1
            while len(vecs) > 1:
                idxs = lane ^ sh
                m = (lane & sh) != 0
                nxt = []
                for q in range(len(vecs) // 2):
                    u, v = vecs[2 * q], vecs[2 * q + 1]
                    gu = u.at[idxs].get(mode="promise_in_bounds")
                    gv = v.at[idxs].get(mode="promise_in_bounds")
                    nxt.append(jnp.where(m, v + gv, u + gu))
                vecs = nxt
                sh *= 2
            sl = pl.ds(k * _CHUNK + g * 16, 16)
            out_v[sl] = AVG_RATING + ub_v[sl] + ib_v[sl] + vecs[0]
            return carry

        lax.fori_loop(0, _GROUPS, group, 0)

    copies = []
    for k in range(_NCHUNK):
        copies.append((
            pltpu.async_copy(ue2_hbm.at[idxh.at[2 * k]], u2.at[k % 2], sem),
            pltpu.async_copy(ie2_hbm.at[idxh.at[2 * k + 1]], i2.at[k % 2], sem),
        ))
        if k > 0:
            for c in copies[k - 1]:
                c.wait()
            chunk_compute(k - 1, u2.at[(k - 1) % 2], i2.at[(k - 1) % 2])
    for c in copies[_NCHUNK - 1]:
        c.wait()
    for c in bias_copies:
        c.wait()
    chunk_compute(_NCHUNK - 1, u2.at[(_NCHUNK - 1) % 2], i2.at[(_NCHUNK - 1) % 2])

    pltpu.sync_copy(out_v, out_hbm.at[pl.ds(base, _BPW)])
    pltpu.sync_copy(ub_v, ub_hbm.at[pl.ds(base, _BPW)])
    pltpu.sync_copy(ib_v, ib_hbm.at[pl.ds(base, _BPW)])


@functools.partial(
    pl.kernel,
    mesh=plsc.VectorSubcoreMesh(core_axis_name="c", subcore_axis_name="s"),
    out_type=(
        jax.ShapeDtypeStruct((BATCH,), jnp.float32),
        jax.ShapeDtypeStruct((BATCH,), jnp.float32),
        jax.ShapeDtypeStruct((BATCH,), jnp.float32),
    ),
    scratch_types=[
        pltpu.VMEM((_NCHUNK, _CHUNK), jnp.int32),
        pltpu.VMEM((_NCHUNK, _CHUNK), jnp.int32),
        pltpu.VMEM((2 * _NCHUNK, _CHUNK), jnp.int32),
        pltpu.VMEM((2, _CHUNK, 128), jnp.float32),
        pltpu.VMEM((2, _CHUNK, 128), jnp.float32),
        pltpu.VMEM((_BPW,), jnp.float32),
        pltpu.VMEM((_BPW,), jnp.float32),
        pltpu.VMEM((_BPW,), jnp.float32),
        pltpu.SemaphoreType.DMA,
        pltpu.SemaphoreType.DMA,
    ],
)
def _svd_sc(*refs):
    _body(*refs)


def kernel(user, item, user_emb, item_emb, user_bias, item_bias):
    ue2 = user_emb.reshape(user_emb.shape[0] // 2, 2 * EMBED_DIM)
    ie2 = item_emb.reshape(item_emb.shape[0] // 2, 2 * EMBED_DIM)
    return _svd_sc(user, item, ue2, ie2, user_bias, item_bias)
